# Initial kernel scaffold; baseline (speedup 1.0000x reference)
import jax
import jax.numpy as jnp
from jax.experimental import pallas as pl


def kernel(preds, targets, masks, img_w, img_h):
    raise NotImplementedError("write your pallas kernel here")


# speedup vs baseline: 62.1475x; 62.1475x over previous
import jax
import jax.numpy as jnp
from jax.experimental import pallas as pl
from jax.experimental.pallas import tpu as pltpu

_SIMOTA_Q = 4
_W_REG = 3.0
_W_CLS = 1.0


def _assign_body(pt_ref, tgt_ref, validf_ref, aux_ref, assigned_ref, matched_ref):
    pt = pt_ref[0]
    tgt = tgt_ref[0]
    validf = validf_ref[0]
    aux = aux_ref[0]
    M = tgt.shape[0]
    N = pt.shape[1]

    labf = aux[:, 0:1]
    mskf = aux[:, 1:2]

    pred_dx = pt[6:78, :]
    tgt_dx = tgt[:, 6:78]

    t3 = tgt_dx[:, :, None]
    p3 = pred_dx[None, :, :]
    v3 = validf[:, :, None]
    lo = jnp.minimum(t3, p3)
    hi = jnp.maximum(t3, p3)
    absd = hi - lo
    absd = jnp.where(v3 > 0, absd, 0.0)
    S = jnp.sum(absd, axis=1)
    ovr = (lo + 15.0) - (hi - 15.0)
    uni = (hi + 15.0) - (lo - 15.0)
    ovr = jnp.where(v3 > 0, ovr, 0.0)
    uni = jnp.where(v3 > 0, uni, 0.0)
    O = jnp.sum(ovr, axis=1)
    U = jnp.sum(uni, axis=1)

    lengths = jnp.maximum(jnp.sum(validf, axis=1, keepdims=True), 1.0)
    dist = S / lengths
    max_d = jnp.maximum(jnp.max(dist), 1e-6)
    d_score = 1.0 - dist / max_d + 0.01

    px = pt[2:3, :]
    py = pt[3:4, :]
    tx = tgt[:, 2:3]
    ty = tgt[:, 3:4]
    xd = px - tx
    yd = py - ty
    xy = jnp.sqrt(xd * xd + yd * yd)
    max_xy = jnp.maximum(jnp.max(xy), 1e-6)
    xy_score = 1.0 - xy / max_xy + 0.01

    pth = pt[4:5, :]
    tth = tgt[:, 4:5]
    th = jnp.abs(pth - tth)
    max_th = jnp.maximum(jnp.max(th), 1e-6)
    th_score = 1.0 - th / max_th + 0.01

    cls_pred = pt[0:2, :]
    p = jax.nn.sigmoid(cls_pred)
    neg = -jnp.log(1.0 - p + 1e-12) * (1.0 - 0.25) * (p * p)
    pos = -jnp.log(p + 1e-12) * 0.25 * ((1.0 - p) * (1.0 - p))
    cdiff = pos - neg
    cls_cost = jnp.where(labf < 1.0, cdiff[0:1, :], cdiff[1:2, :])

    prod = d_score * xy_score * th_score
    cost = -(prod * prod) * _W_REG + cls_cost * _W_CLS
    cost = jnp.where(mskf > 0, cost, 100000.0)

    iou = O / (U + 1e-9)
    iou = jnp.where(mskf > 0, iou, 0.0)
    ious = jnp.maximum(iou, 0.0)

    iota_n = jax.lax.broadcasted_iota(jnp.int32, (M, N), 1)

    iw = ious
    acc = jnp.zeros((M, 1), jnp.float32)
    for _ in range(_SIMOTA_Q):
        v = jnp.max(iw, axis=1, keepdims=True)
        ii = jnp.min(jnp.where(iw == v, iota_n, N), axis=1, keepdims=True)
        acc = acc + v
        iw = jnp.where(iota_n == ii, -1.0, iw)
    ks = jnp.clip(acc.astype(jnp.int32), 1, N)
    kidx = ks - 1

    cw = cost
    vals = []
    idxs = []
    for _ in range(_SIMOTA_Q):
        v = jnp.min(cw, axis=1, keepdims=True)
        ii = jnp.min(jnp.where(cw == v, iota_n, N), axis=1, keepdims=True)
        vals.append(v)
        idxs.append(ii)
        cw = jnp.where(iota_n == ii, jnp.inf, cw)
    c_th = jnp.where(kidx == 0, vals[0],
           jnp.where(kidx == 1, vals[1],
           jnp.where(kidx == 2, vals[2], vals[3])))
    i_th = jnp.where(kidx == 0, idxs[0],
           jnp.where(kidx == 1, idxs[1],
           jnp.where(kidx == 2, idxs[2], idxs[3])))

    matching = (cost < c_th) | ((cost == c_th) & (iota_n <= i_th))
    matching = matching & (mskf > 0)

    matched_gt = jnp.sum(matching.astype(jnp.float32), axis=0, keepdims=True)
    cost_inf = jnp.where(matching, cost, jnp.inf)
    rowmin = jnp.min(cost_inf, axis=0, keepdims=True)
    iota_m = jax.lax.broadcasted_iota(jnp.int32, (M, N), 0)
    midx = jnp.min(jnp.where(cost_inf == rowmin, iota_m, M), axis=0, keepdims=True)
    assigned = matched_gt > 0.0
    matched = jnp.where(assigned, midx, -1)

    assigned_ref[0] = assigned.astype(jnp.int32)
    matched_ref[0] = matched


def kernel(preds, targets, masks, img_w, img_h):
    B, N, _ = preds.shape
    M = targets.shape[1]
    preds_t = jnp.swapaxes(preds, 1, 2)
    tgt_dx = targets[..., 6:]
    validf = ((tgt_dx >= 0) & (tgt_dx < img_w)).astype(jnp.float32)
    aux = jnp.zeros((B, M, 8), jnp.float32)
    aux = aux.at[..., 0].set(targets[..., 1])
    aux = aux.at[..., 1].set(masks.astype(jnp.float32))

    assigned_i, matched = pl.pallas_call(
        _assign_body,
        grid=(B,),
        in_specs=[
            pl.BlockSpec((1, 78, N), lambda b: (b, 0, 0)),
            pl.BlockSpec((1, M, 78), lambda b: (b, 0, 0)),
            pl.BlockSpec((1, M, 72), lambda b: (b, 0, 0)),
            pl.BlockSpec((1, M, 8), lambda b: (b, 0, 0)),
        ],
        out_specs=[
            pl.BlockSpec((1, 1, N), lambda b: (b, 0, 0)),
            pl.BlockSpec((1, 1, N), lambda b: (b, 0, 0)),
        ],
        out_shape=[
            jax.ShapeDtypeStruct((B, 1, N), jnp.int32),
            jax.ShapeDtypeStruct((B, 1, N), jnp.int32),
        ],
        compiler_params=pltpu.CompilerParams(
            dimension_semantics=("arbitrary",),
        ),
    )(preds_t, targets, validf, aux)

    assigned = assigned_i.reshape(B, N).astype(jnp.bool_)
    return assigned, matched.reshape(B, N)
